# baseline (speedup 1.0000x reference)
import jax
import jax.numpy as jnp
from jax import lax
from jax.experimental import pallas as pl
from jax.experimental.pallas import tpu as pltpu
from jax.experimental.pallas import tpu_sc as plsc

_B = 16384
_D = 32
_N = 1000000
_NC = 2
_NS = 16
_NW = _NC * _NS
_BPW = _B // _NW
_GW = 128
_CH = 16


def _body(flat_hbm, src_hbm, dst_hbm, out_hbm,
          sidx_v, didx_v, sgi_v, scols_v, dcols_v, out_v, ssem, dsem):
    wid = lax.axis_index("s") * _NC + lax.axis_index("c")
    base = wid * _BPW

    pltpu.sync_copy(src_hbm.at[pl.ds(base, _BPW)], sidx_v)
    pltpu.sync_copy(dst_hbm.at[pl.ds(base, _BPW)], didx_v)

    def fire(d, carry):
        row = flat_hbm.at[pl.ds(d * _N, _N)]
        for j in range(_BPW // _GW):
            sl = pl.ds(j * _GW, _GW)
            pltpu.async_copy(row.at[sidx_v.at[sl]], scols_v.at[d, sl], ssem)
            pltpu.async_copy(row.at[didx_v.at[sl]], dcols_v.at[d, sl], dsem)
        return carry

    lax.fori_loop(0, _D, fire, 0)

    pltpu.make_async_copy(
        flat_hbm.at[pl.ds(0, _D * _BPW)], sgi_v, ssem).wait()
    pltpu.make_async_copy(
        flat_hbm.at[pl.ds(0, _D * _BPW)], sgi_v, dsem).wait()

    def chunk(c, carry):
        sl = pl.ds(c * _CH, _CH)
        acc = scols_v[0, sl] * dcols_v[0, sl]
        for d in range(1, _D):
            acc = acc + scols_v[d, sl] * dcols_v[d, sl]
        out_v[sl] = 1.0 / (1.0 + jnp.exp(-acc))
        return carry

    lax.fori_loop(0, _BPW // _CH, chunk, 0)

    pltpu.sync_copy(out_v, out_hbm.at[pl.ds(base, _BPW)])


def kernel(embedding_user, src, dst):
    mesh = plsc.VectorSubcoreMesh(core_axis_name="c", subcore_axis_name="s")
    k = pl.kernel(
        _body,
        mesh=mesh,
        out_type=jax.ShapeDtypeStruct((_B,), jnp.float32),
        scratch_types=[
            pltpu.VMEM((_BPW,), jnp.int32),
            pltpu.VMEM((_BPW,), jnp.int32),
            pltpu.VMEM((_D * _BPW,), jnp.float32),
            pltpu.VMEM((_D, _BPW), jnp.float32),
            pltpu.VMEM((_D, _BPW), jnp.float32),
            pltpu.VMEM((_BPW,), jnp.float32),
            pltpu.SemaphoreType.DMA,
            pltpu.SemaphoreType.DMA,
        ],
    )
    return k(embedding_user.T.reshape(-1), src, dst)


# speedup vs baseline: 4.6898x; 4.6898x over previous
import jax
import jax.numpy as jnp
from jax import lax
from jax.experimental import pallas as pl
from jax.experimental.pallas import tpu as pltpu
from jax.experimental.pallas import tpu_sc as plsc

_B = 16384
_D = 32
_N = 1000000
_NC = 2
_NS = 16
_NW = _NC * _NS
_BPW = _B // _NW
_GW = 128
_CH = 16


def _body(flat_hbm, src_hbm, dst_hbm, out_hbm,
          sidx_v, didx_v, sgi_v, dgi_v, scols_v, dcols_v, drain_v, out_v,
          ssem, dsem):
    wid = lax.axis_index("s") * _NC + lax.axis_index("c")
    base = wid * _BPW

    pltpu.sync_copy(src_hbm.at[pl.ds(base, _BPW)], sidx_v)
    pltpu.sync_copy(dst_hbm.at[pl.ds(base, _BPW)], didx_v)

    def mkidx(c, carry):
        sl = pl.ds(c * _CH, _CH)
        sv = sidx_v[sl] << 5
        dv = didx_v[sl] << 5
        for d in range(_D):
            sgi_v[d, sl] = sv + d
            dgi_v[d, sl] = dv + d
        return carry

    lax.fori_loop(0, _BPW // _CH, mkidx, 0)

    def fire(d, carry):
        for j in range(_BPW // _GW):
            sl = pl.ds(j * _GW, _GW)
            pltpu.async_copy(
                flat_hbm.at[sgi_v.at[d, sl]], scols_v.at[d, sl], ssem)
            pltpu.async_copy(
                flat_hbm.at[dgi_v.at[d, sl]], dcols_v.at[d, sl], dsem)
        return carry

    lax.fori_loop(0, _D, fire, 0)

    pltpu.make_async_copy(
        flat_hbm.at[pl.ds(0, _D * _BPW)], drain_v, ssem).wait()
    pltpu.make_async_copy(
        flat_hbm.at[pl.ds(0, _D * _BPW)], drain_v, dsem).wait()

    def chunk(c, carry):
        sl = pl.ds(c * _CH, _CH)
        acc = scols_v[0, sl] * dcols_v[0, sl]
        for d in range(1, _D):
            acc = acc + scols_v[d, sl] * dcols_v[d, sl]
        out_v[sl] = 1.0 / (1.0 + jnp.exp(-acc))
        return carry

    lax.fori_loop(0, _BPW // _CH, chunk, 0)

    pltpu.sync_copy(out_v, out_hbm.at[pl.ds(base, _BPW)])


def kernel(embedding_user, src, dst):
    mesh = plsc.VectorSubcoreMesh(core_axis_name="c", subcore_axis_name="s")
    k = pl.kernel(
        _body,
        mesh=mesh,
        out_type=jax.ShapeDtypeStruct((_B,), jnp.float32),
        scratch_types=[
            pltpu.VMEM((_BPW,), jnp.int32),
            pltpu.VMEM((_BPW,), jnp.int32),
            pltpu.VMEM((_D, _BPW), jnp.int32),
            pltpu.VMEM((_D, _BPW), jnp.int32),
            pltpu.VMEM((_D, _BPW), jnp.float32),
            pltpu.VMEM((_D, _BPW), jnp.float32),
            pltpu.VMEM((_D * _BPW,), jnp.float32),
            pltpu.VMEM((_BPW,), jnp.float32),
            pltpu.SemaphoreType.DMA,
            pltpu.SemaphoreType.DMA,
        ],
    )
    return k(embedding_user.reshape(-1), src, dst)


# speedup vs baseline: 7.3539x; 1.5681x over previous
import jax
import jax.numpy as jnp
from jax import lax
from jax.experimental import pallas as pl
from jax.experimental.pallas import tpu as pltpu
from jax.experimental.pallas import tpu_sc as plsc

_B = 16384
_D = 32
_N = 1000000
_NC = 2
_NS = 16
_NW = _NC * _NS
_BPW = _B // _NW
_GW = 128
_CH = 16

_W = 1024
_NFULL = _N // _W
_REM0 = _NFULL * _W
_REM1 = 999936
_TAIL = _N - _REM1


def _flatten_body(tableT_hbm, tailT_hbm, flat_hbm, buf_v, stage_v, tail_v,
                  sem):
    wid = lax.axis_index("s") * _NC + lax.axis_index("c")

    nb = jnp.where(wid < _NFULL % _NW, _NFULL // _NW + 1, _NFULL // _NW)

    def _detile(width, lane):
        def per_d(d, carry):
            for g in range(width // 16):
                stage_v[pl.ds(d * width + g * 16, 16)] = (
                    buf_v[d, pl.ds(g * 16, 16)])
            pltpu.async_copy(
                stage_v.at[pl.ds(d * width, width)],
                flat_hbm.at[pl.ds(pl.multiple_of(d * _N + lane, 8), width)],
                sem)
            return carry

        lax.fori_loop(0, _D, per_d, 0)
        pltpu.make_async_copy(
            flat_hbm.at[pl.ds(0, _D * width)],
            stage_v.at[pl.ds(0, _D * width)], sem).wait()

    def block(b, carry):
        lane = pl.multiple_of((wid + b * _NW) * _W, 128)
        pltpu.sync_copy(tableT_hbm.at[:, pl.ds(lane, _W)], buf_v)
        _detile(_W, lane)
        return carry

    lax.fori_loop(0, nb, block, 0)

    @pl.when(wid == 0)
    def _rem():
        pltpu.sync_copy(
            tableT_hbm.at[:, pl.ds(_REM0, _REM1 - _REM0)],
            buf_v.at[:, pl.ds(0, _REM1 - _REM0)])
        _detile(_REM1 - _REM0, _REM0)

    @pl.when(wid == 1)
    def _tail():
        pltpu.sync_copy(tailT_hbm, tail_v)
        copies = []
        for d in range(_D):
            copies.append(pltpu.async_copy(
                tail_v.at[pl.ds(d * _TAIL, _TAIL)],
                flat_hbm.at[pl.ds(d * _N + _REM1, _TAIL)], sem))
        for cp in copies:
            cp.wait()


def _score_body(flat_hbm, src_hbm, dst_hbm, out_hbm,
                sidx_v, didx_v, scols_v, dcols_v, drain_v, out_v,
                ssem, dsem):
    wid = lax.axis_index("s") * _NC + lax.axis_index("c")
    base = wid * _BPW

    pltpu.sync_copy(src_hbm.at[pl.ds(base, _BPW)], sidx_v)
    pltpu.sync_copy(dst_hbm.at[pl.ds(base, _BPW)], didx_v)

    def fire(d, carry):
        row = flat_hbm.at[pl.ds(d * _N, _N)]
        for j in range(_BPW // _GW):
            sl = pl.ds(j * _GW, _GW)
            pltpu.async_copy(row.at[sidx_v.at[sl]], scols_v.at[d, sl], ssem)
            pltpu.async_copy(row.at[didx_v.at[sl]], dcols_v.at[d, sl], dsem)
        return carry

    lax.fori_loop(0, _D, fire, 0)

    pltpu.make_async_copy(
        flat_hbm.at[pl.ds(0, _D * _BPW)], drain_v, ssem).wait()
    pltpu.make_async_copy(
        flat_hbm.at[pl.ds(0, _D * _BPW)], drain_v, dsem).wait()

    def chunk(c, carry):
        sl = pl.ds(c * _CH, _CH)
        acc = scols_v[0, sl] * dcols_v[0, sl]
        for d in range(1, _D):
            acc = acc + scols_v[d, sl] * dcols_v[d, sl]
        out_v[sl] = 1.0 / (1.0 + jnp.exp(-acc))
        return carry

    lax.fori_loop(0, _BPW // _CH, chunk, 0)

    pltpu.sync_copy(out_v, out_hbm.at[pl.ds(base, _BPW)])


def kernel(embedding_user, src, dst):
    mesh = plsc.VectorSubcoreMesh(core_axis_name="c", subcore_axis_name="s")
    flatten = pl.kernel(
        _flatten_body,
        mesh=mesh,
        out_type=jax.ShapeDtypeStruct((_D * _N,), jnp.float32),
        scratch_types=[
            pltpu.VMEM((_D, _W), jnp.float32),
            pltpu.VMEM((_D * _W,), jnp.float32),
            pltpu.VMEM((_D * _TAIL,), jnp.float32),
            pltpu.SemaphoreType.DMA,
        ],
    )
    score = pl.kernel(
        _score_body,
        mesh=mesh,
        out_type=jax.ShapeDtypeStruct((_B,), jnp.float32),
        scratch_types=[
            pltpu.VMEM((_BPW,), jnp.int32),
            pltpu.VMEM((_BPW,), jnp.int32),
            pltpu.VMEM((_D, _BPW), jnp.float32),
            pltpu.VMEM((_D, _BPW), jnp.float32),
            pltpu.VMEM((_D * _BPW,), jnp.float32),
            pltpu.VMEM((_BPW,), jnp.float32),
            pltpu.SemaphoreType.DMA,
            pltpu.SemaphoreType.DMA,
        ],
    )
    tailT = embedding_user[_REM1:, :].T.reshape(-1)
    flat = flatten(embedding_user.T, tailT)
    return score(flat, src, dst)


# speedup vs baseline: 7.8584x; 1.0686x over previous
import jax
import jax.numpy as jnp
from jax import lax
from jax.experimental import pallas as pl
from jax.experimental.pallas import tpu as pltpu
from jax.experimental.pallas import tpu_sc as plsc

_B = 16384
_D = 32
_NC = 2
_NS = 16
_NW = _NC * _NS
_BPW = _B // _NW
_CH = 16


def _body(table_hbm, src_hbm, dst_hbm, out_hbm,
          sidx_v, didx_v, srows_v, drows_v, out_v, sem):
    wid = lax.axis_index("s") * _NC + lax.axis_index("c")
    base = wid * _BPW

    pltpu.sync_copy(src_hbm.at[pl.ds(base, _BPW)], sidx_v)
    pltpu.sync_copy(dst_hbm.at[pl.ds(base, _BPW)], didx_v)

    lanes = lax.iota(jnp.int32, 16)
    perm = {h: lanes ^ h for h in (1, 2, 4, 8)}
    bit = {h: (lanes & h) != 0 for h in (1, 2, 4, 8)}

    def _take(x, idx):
        return jnp.take_along_axis(x, idx, axis=0, mode="promise_in_bounds")

    def chunk(c, carry):
        sv = sidx_v[pl.ds(c * _CH, _CH)]
        dv = didx_v[pl.ds(c * _CH, _CH)]
        copies = []
        for r in range(_CH):
            copies.append(pltpu.async_copy(
                table_hbm.at[pl.ds(sv[r], 1)], srows_v.at[pl.ds(r, 1)], sem))
            copies.append(pltpu.async_copy(
                table_hbm.at[pl.ds(dv[r], 1)], drows_v.at[pl.ds(r, 1)], sem))
        for cp in copies:
            cp.wait()
        regs = []
        for r in range(_CH):
            lo = srows_v[r, pl.ds(0, 16)] * drows_v[r, pl.ds(0, 16)]
            hi = srows_v[r, pl.ds(16, 16)] * drows_v[r, pl.ds(16, 16)]
            regs.append(lo + hi)
        for h in (1, 2, 4, 8):
            nxt = []
            for i in range(0, len(regs), 2):
                u, v = regs[i], regs[i + 1]
                t1 = jnp.where(bit[h], v, u)
                t2 = _take(jnp.where(bit[h], u, v), perm[h])
                nxt.append(t1 + t2)
            regs = nxt
        acc = regs[0]
        out_v[pl.ds(c * _CH, _CH)] = 1.0 / (1.0 + jnp.exp(-acc))
        return carry

    lax.fori_loop(0, _BPW // _CH, chunk, 0)

    pltpu.sync_copy(out_v, out_hbm.at[pl.ds(base, _BPW)])


def kernel(embedding_user, src, dst):
    mesh = plsc.VectorSubcoreMesh(core_axis_name="c", subcore_axis_name="s")
    k = pl.kernel(
        _body,
        mesh=mesh,
        out_type=jax.ShapeDtypeStruct((_B,), jnp.float32),
        scratch_types=[
            pltpu.VMEM((_BPW,), jnp.int32),
            pltpu.VMEM((_BPW,), jnp.int32),
            pltpu.VMEM((_CH, _D), jnp.float32),
            pltpu.VMEM((_CH, _D), jnp.float32),
            pltpu.VMEM((_BPW,), jnp.float32),
            pltpu.SemaphoreType.DMA,
        ],
    )
    return k(embedding_user, src, dst)


# speedup vs baseline: 8.2006x; 1.0435x over previous
import jax
import jax.numpy as jnp
from jax import lax
from jax.experimental import pallas as pl
from jax.experimental.pallas import tpu as pltpu
from jax.experimental.pallas import tpu_sc as plsc

_B = 16384
_D = 32
_NC = 2
_NS = 16
_NW = _NC * _NS
_BPW = _B // _NW
_CH = 64


def _body(table_hbm, src_hbm, dst_hbm, out_hbm,
          sidx_v, didx_v, srows_v, drows_v, out_v, sem):
    wid = lax.axis_index("s") * _NC + lax.axis_index("c")
    base = wid * _BPW

    pltpu.sync_copy(src_hbm.at[pl.ds(base, _BPW)], sidx_v)
    pltpu.sync_copy(dst_hbm.at[pl.ds(base, _BPW)], didx_v)

    lanes = lax.iota(jnp.int32, 16)
    perm = {h: lanes ^ h for h in (1, 2, 4, 8)}
    bit = {h: (lanes & h) != 0 for h in (1, 2, 4, 8)}

    def _take(x, idx):
        return jnp.take_along_axis(x, idx, axis=0, mode="promise_in_bounds")

    def chunk(c, carry):
        copies = []
        for g in range(_CH // 16):
            sv = sidx_v[pl.ds(c * _CH + g * 16, 16)]
            dv = didx_v[pl.ds(c * _CH + g * 16, 16)]
            for r in range(16):
                i = g * 16 + r
                copies.append(pltpu.async_copy(
                    table_hbm.at[pl.ds(sv[r], 1)],
                    srows_v.at[pl.ds(i, 1)], sem))
                copies.append(pltpu.async_copy(
                    table_hbm.at[pl.ds(dv[r], 1)],
                    drows_v.at[pl.ds(i, 1)], sem))
        for cp in copies:
            cp.wait()
        for g in range(_CH // 16):
            regs = []
            for r in range(16):
                i = g * 16 + r
                lo = srows_v[i, pl.ds(0, 16)] * drows_v[i, pl.ds(0, 16)]
                hi = srows_v[i, pl.ds(16, 16)] * drows_v[i, pl.ds(16, 16)]
                regs.append(lo + hi)
            for h in (1, 2, 4, 8):
                nxt = []
                for k in range(0, len(regs), 2):
                    u, v = regs[k], regs[k + 1]
                    t1 = jnp.where(bit[h], v, u)
                    t2 = _take(jnp.where(bit[h], u, v), perm[h])
                    nxt.append(t1 + t2)
                regs = nxt
            acc = regs[0]
            out_v[pl.ds(c * _CH + g * 16, 16)] = 1.0 / (1.0 + jnp.exp(-acc))
        return carry

    lax.fori_loop(0, _BPW // _CH, chunk, 0)

    pltpu.sync_copy(out_v, out_hbm.at[pl.ds(base, _BPW)])


def kernel(embedding_user, src, dst):
    mesh = plsc.VectorSubcoreMesh(core_axis_name="c", subcore_axis_name="s")
    k = pl.kernel(
        _body,
        mesh=mesh,
        out_type=jax.ShapeDtypeStruct((_B,), jnp.float32),
        scratch_types=[
            pltpu.VMEM((_BPW,), jnp.int32),
            pltpu.VMEM((_BPW,), jnp.int32),
            pltpu.VMEM((_CH, _D), jnp.float32),
            pltpu.VMEM((_CH, _D), jnp.float32),
            pltpu.VMEM((_BPW,), jnp.float32),
            pltpu.SemaphoreType.DMA,
        ],
    )
    return k(embedding_user, src, dst)
